# baseline (speedup 1.0000x reference)
import functools
import math

import jax
import jax.numpy as jnp
from jax import lax
from jax.experimental import pallas as pl
from jax.experimental.pallas import tpu as pltpu
from jax.experimental.pallas import tpu_sc as plsc

_SMOOTHING = 0.1
_V = 32000
_N = 2048
_EPS = _SMOOTHING / (_V - 2)
_K = (1.0 - _SMOOTHING) * math.log(1.0 - _SMOOTHING) + _SMOOTHING * math.log(_EPS)
_COEF = (1.0 - _SMOOTHING) - _EPS

_NTC = 1024
_NSC = _N - _NTC
_ROWS = 64
_NBLK = _NTC // _ROWS
_NBUF = 4
_NSTEP = _NBLK // _NBUF
_LANES = 16
_NW = 32
_PER_W = _N // _NW
_RPS = _NSC // _NW
_CHUNK = 640


def _tc_body(t_ref, x_hbm, out_ref, buf, sem):
    i = pl.program_id(0)

    def _copy(blk, b):
        return pltpu.make_async_copy(
            x_hbm.at[pl.ds(blk * _ROWS, _ROWS), :], buf.at[b], sem.at[b]
        )

    @pl.when(i == 0)
    def _():
        out_ref[...] = jnp.zeros((1, 1), jnp.float32)
        for b in range(_NBUF):
            _copy(b, b).start()

    cols = lax.broadcasted_iota(jnp.int32, (_ROWS, _V), 1)
    for b in range(_NBUF):
        blk = i * _NBUF + b
        _copy(blk, b).wait()
        xs = buf[b]
        tcol = t_ref[pl.ds(blk * _ROWS, _ROWS), :]
        mask = (tcol != 0).astype(jnp.float32)
        total = jnp.sum(xs * mask)
        col0 = jnp.sum(xs[:, 0:1] * mask)
        cnt = jnp.sum(mask)
        gsum = jnp.sum(jnp.where(cols == tcol, xs, 0.0) * mask)
        part = jnp.reshape(
            _K * cnt + _EPS * col0 - _EPS * total - _COEF * gsum, (1, 1)
        )
        out_ref[...] = out_ref[...] + part
        nxt = blk + _NBUF

        @pl.when(nxt < _NBLK)
        def _():
            _copy(nxt, b).start()


def _tc_sweep(x2, t2):
    return pl.pallas_call(
        _tc_body,
        grid=(_NSTEP,),
        in_specs=[
            pl.BlockSpec((_N, 1), lambda i: (0, 0)),
            pl.BlockSpec(memory_space=pltpu.MemorySpace.HBM),
        ],
        out_specs=pl.BlockSpec((1, 1), lambda i: (0, 0)),
        out_shape=jax.ShapeDtypeStruct((1, 1), jnp.float32),
        scratch_shapes=[
            pltpu.VMEM((_NBUF, _ROWS, _V), jnp.float32),
            pltpu.SemaphoreType.DMA((_NBUF,)),
        ],
    )(t2, x2)


def _lane_gather(v, idx):
    return v.at[idx].get(mode="promise_in_bounds")


def _all_lane_sum(v, lanes):
    for sh in (8, 4, 2, 1):
        v = v + _lane_gather(v, (lanes + sh) & 15)
    return v


def _row_sum(buf):
    def body(k, acc):
        s = None
        kbase = k * _CHUNK
        for m in range(_CHUNK // _LANES):
            v = buf[pl.ds(kbase + m * _LANES, _LANES)]
            s = v if s is None else s + v
        return acc + s

    return lax.fori_loop(0, _V // _CHUNK, body, jnp.zeros((_LANES,), jnp.float32))


def _sc_body(t_hbm, x_hbm, out_hbm, tgtd_v, buf0, buf1, res_v, sem0, sem1):
    wid = lax.axis_index("s") * 2 + lax.axis_index("c")
    lanes = lax.iota(jnp.int32, _LANES)

    dbase = _NTC + wid * _RPS
    pltpu.sync_copy(t_hbm.at[pl.ds(dbase, _RPS)], tgtd_v)

    bufs = (buf0, buf1)
    sems = (sem0, sem1)

    def _rcopy(r, p):
        return pltpu.make_async_copy(x_hbm.at[dbase + r], bufs[p], sems[p])

    _rcopy(0, 0).start()
    _rcopy(1, 1).start()

    dense = jnp.zeros((_LANES,), jnp.float32)
    x0s = jnp.zeros((_LANES,), jnp.float32)
    cnts = jnp.zeros((_LANES,), jnp.float32)
    gath = jnp.zeros((_LANES,), jnp.float32)
    for g in range(_RPS // _LANES):
        t16 = tgtd_v[pl.ds(g * _LANES, _LANES)]
        svec = jnp.zeros((_LANES,), jnp.float32)
        x0vec = jnp.zeros((_LANES,), jnp.float32)
        gvec = jnp.zeros((_LANES,), jnp.float32)
        for j in range(_LANES):
            r = g * _LANES + j
            p = j & 1
            tr = t16[j]
            _rcopy(r, p).wait()
            head = bufs[p][pl.ds(0, _LANES)]
            win = bufs[p][pl.ds((tr >> 4) << 4, _LANES)]
            acc = _row_sum(bufs[p])
            if r + 2 < _RPS:
                _rcopy(r + 2, p).start()
            s = _all_lane_sum(acc, lanes)
            x0 = _lane_gather(head, jnp.zeros((_LANES,), jnp.int32))
            gval = _lane_gather(win, jnp.zeros((_LANES,), jnp.int32) + (tr & 15))
            svec = jnp.where(lanes == j, s, svec)
            x0vec = jnp.where(lanes == j, x0, x0vec)
            gvec = jnp.where(lanes == j, gval, gvec)
        m = t16 != 0
        dense = dense + jnp.where(m, svec, 0.0)
        x0s = x0s + jnp.where(m, x0vec, 0.0)
        cnts = cnts + jnp.where(m, 1.0, 0.0)
        gath = gath + jnp.where(m, gvec, 0.0)

    res_v[...] = _K * cnts + _EPS * x0s - _EPS * dense - _COEF * gath
    pltpu.sync_copy(res_v, out_hbm.at[wid])


@functools.lru_cache(maxsize=None)
def _sc_kernel():
    return pl.kernel(
        _sc_body,
        out_type=jax.ShapeDtypeStruct((_NW, _LANES), jnp.float32),
        mesh=plsc.VectorSubcoreMesh(core_axis_name="c", subcore_axis_name="s"),
        scratch_types=[
            pltpu.VMEM((_RPS,), jnp.int32),
            pltpu.VMEM((_V,), jnp.float32),
            pltpu.VMEM((_V,), jnp.float32),
            pltpu.VMEM((_LANES,), jnp.float32),
            pltpu.SemaphoreType.DMA,
            pltpu.SemaphoreType.DMA,
        ],
    )


@jax.jit
def kernel(x, target):
    t = target.reshape(-1).astype(jnp.int32)
    x2 = x.reshape(_N, _V)
    sc_part = _sc_kernel()(t, x2)
    tc_part = _tc_sweep(x2, t.reshape(_N, 1))
    return tc_part[0, 0] + jnp.sum(sc_part)


# speedup vs baseline: 1.0343x; 1.0343x over previous
import functools
import math

import jax
import jax.numpy as jnp
from jax import lax
from jax.experimental import pallas as pl
from jax.experimental.pallas import tpu as pltpu
from jax.experimental.pallas import tpu_sc as plsc

_SMOOTHING = 0.1
_V = 32000
_N = 2048
_EPS = _SMOOTHING / (_V - 2)
_K = (1.0 - _SMOOTHING) * math.log(1.0 - _SMOOTHING) + _SMOOTHING * math.log(_EPS)
_COEF = (1.0 - _SMOOTHING) - _EPS

_NTC = 1152
_NSC = _N - _NTC
_ROWS = 64
_NBLK = _NTC // _ROWS
_NBUF = 3
_NSTEP = _NBLK // _NBUF
_LANES = 16
_NW = 32
_RPS = _NSC // _NW
_NGRP = -(-_RPS // _LANES)
_CHUNK = 256


def _tc_body(t_ref, x_hbm, out_ref, buf, sem):
    i = pl.program_id(0)

    def _copy(blk, b):
        return pltpu.make_async_copy(
            x_hbm.at[pl.ds(blk * _ROWS, _ROWS), :], buf.at[b], sem.at[b]
        )

    @pl.when(i == 0)
    def _():
        out_ref[...] = jnp.zeros((1, 1), jnp.float32)
        for b in range(_NBUF):
            _copy(b, b).start()

    cols = lax.broadcasted_iota(jnp.int32, (_ROWS, _V), 1)
    for b in range(_NBUF):
        blk = i * _NBUF + b
        _copy(blk, b).wait()
        xs = buf[b]
        tcol = t_ref[pl.ds(blk * _ROWS, _ROWS), :]
        mask = (tcol != 0).astype(jnp.float32)
        total = jnp.sum(xs * mask)
        col0 = jnp.sum(xs[:, 0:1] * mask)
        cnt = jnp.sum(mask)
        gsum = jnp.sum(jnp.where(cols == tcol, xs, 0.0) * mask)
        part = jnp.reshape(
            _K * cnt + _EPS * col0 - _EPS * total - _COEF * gsum, (1, 1)
        )
        out_ref[...] = out_ref[...] + part
        nxt = blk + _NBUF

        @pl.when(nxt < _NBLK)
        def _():
            _copy(nxt, b).start()


def _tc_sweep(x2, t2):
    return pl.pallas_call(
        _tc_body,
        grid=(_NSTEP,),
        in_specs=[
            pl.BlockSpec((_N, 1), lambda i: (0, 0)),
            pl.BlockSpec(memory_space=pltpu.MemorySpace.HBM),
        ],
        out_specs=pl.BlockSpec((1, 1), lambda i: (0, 0)),
        out_shape=jax.ShapeDtypeStruct((1, 1), jnp.float32),
        scratch_shapes=[
            pltpu.VMEM((_NBUF, _ROWS, _V), jnp.float32),
            pltpu.SemaphoreType.DMA((_NBUF,)),
        ],
    )(t2, x2)


def _lane_gather(v, idx):
    return v.at[idx].get(mode="promise_in_bounds")


def _all_lane_sum(v, lanes):
    for sh in (8, 4, 2, 1):
        v = v + _lane_gather(v, (lanes + sh) & 15)
    return v


def _row_sum(buf):
    def body(k, acc):
        s = None
        kbase = k * _CHUNK
        for m in range(_CHUNK // _LANES):
            v = buf[pl.ds(kbase + m * _LANES, _LANES)]
            s = v if s is None else s + v
        return acc + s

    return lax.fori_loop(0, _V // _CHUNK, body, jnp.zeros((_LANES,), jnp.float32))


def _sc_body(t_hbm, x_hbm, out_hbm, tgtd_v, buf0, buf1, res_v, sem0, sem1):
    wid = lax.axis_index("s") * 2 + lax.axis_index("c")
    lanes = lax.iota(jnp.int32, _LANES)

    dbase = _NTC + wid * _RPS
    ofs = dbase & 7
    pltpu.sync_copy(
        t_hbm.at[pl.ds(pl.multiple_of(dbase - ofs, 8), _NGRP * _LANES)],
        tgtd_v.at[pl.ds(0, _NGRP * _LANES)],
    )

    bufs = (buf0, buf1)
    sems = (sem0, sem1)

    def _rcopy(r, p):
        return pltpu.make_async_copy(x_hbm.at[dbase + r], bufs[p], sems[p])

    _rcopy(0, 0).start()
    _rcopy(1, 1).start()

    dense = jnp.zeros((_LANES,), jnp.float32)
    x0s = jnp.zeros((_LANES,), jnp.float32)
    cnts = jnp.zeros((_LANES,), jnp.float32)
    gath = jnp.zeros((_LANES,), jnp.float32)
    for g in range(_NGRP):
        jmax = min(_LANES, _RPS - g * _LANES)
        t16 = tgtd_v[pl.ds(ofs + g * _LANES, _LANES)]
        svec = jnp.zeros((_LANES,), jnp.float32)
        x0vec = jnp.zeros((_LANES,), jnp.float32)
        gvec = jnp.zeros((_LANES,), jnp.float32)
        for j in range(jmax):
            r = g * _LANES + j
            p = j & 1
            tr = t16[j]
            _rcopy(r, p).wait()
            head = bufs[p][pl.ds(0, _LANES)]
            win = bufs[p][pl.ds((tr >> 4) << 4, _LANES)]
            acc = _row_sum(bufs[p])
            if r + 2 < _RPS:
                _rcopy(r + 2, p).start()
            s = _all_lane_sum(acc, lanes)
            x0 = _lane_gather(head, jnp.zeros((_LANES,), jnp.int32))
            gval = _lane_gather(win, jnp.zeros((_LANES,), jnp.int32) + (tr & 15))
            svec = jnp.where(lanes == j, s, svec)
            x0vec = jnp.where(lanes == j, x0, x0vec)
            gvec = jnp.where(lanes == j, gval, gvec)
        m = (t16 != 0) & (lanes < jmax)
        dense = dense + jnp.where(m, svec, 0.0)
        x0s = x0s + jnp.where(m, x0vec, 0.0)
        cnts = cnts + jnp.where(m, 1.0, 0.0)
        gath = gath + jnp.where(m, gvec, 0.0)

    res_v[...] = _K * cnts + _EPS * x0s - _EPS * dense - _COEF * gath
    pltpu.sync_copy(res_v, out_hbm.at[wid])


@functools.lru_cache(maxsize=None)
def _sc_kernel():
    return pl.kernel(
        _sc_body,
        out_type=jax.ShapeDtypeStruct((_NW, _LANES), jnp.float32),
        mesh=plsc.VectorSubcoreMesh(core_axis_name="c", subcore_axis_name="s"),
        scratch_types=[
            pltpu.VMEM((_NGRP * _LANES + 8,), jnp.int32),
            pltpu.VMEM((_V,), jnp.float32),
            pltpu.VMEM((_V,), jnp.float32),
            pltpu.VMEM((_LANES,), jnp.float32),
            pltpu.SemaphoreType.DMA,
            pltpu.SemaphoreType.DMA,
        ],
    )


@jax.jit
def kernel(x, target):
    t = target.reshape(-1).astype(jnp.int32)
    x2 = x.reshape(_N, _V)
    sc_part = _sc_kernel()(t, x2)
    tc_part = _tc_sweep(x2, t.reshape(_N, 1))
    return tc_part[0, 0] + jnp.sum(sc_part)


# speedup vs baseline: 1.0369x; 1.0025x over previous
import functools
import math

import jax
import jax.numpy as jnp
from jax import lax
from jax.experimental import pallas as pl
from jax.experimental.pallas import tpu as pltpu
from jax.experimental.pallas import tpu_sc as plsc

_SMOOTHING = 0.1
_V = 32000
_N = 2048
_EPS = _SMOOTHING / (_V - 2)
_K = (1.0 - _SMOOTHING) * math.log(1.0 - _SMOOTHING) + _SMOOTHING * math.log(_EPS)
_COEF = (1.0 - _SMOOTHING) - _EPS

_NTC = 1152
_NSC = _N - _NTC
_ROWS = 64
_NBLK = _NTC // _ROWS
_NBUF = 6
_NSTEP = _NBLK // _NBUF
_LANES = 16
_NW = 32
_RPS = _NSC // _NW
_NGRP = -(-_RPS // _LANES)
_CHUNK = 256


def _tc_body(t_ref, x_hbm, out_ref, buf, sem):
    i = pl.program_id(0)

    def _copy(blk, b):
        return pltpu.make_async_copy(
            x_hbm.at[pl.ds(blk * _ROWS, _ROWS), :], buf.at[b], sem.at[b]
        )

    @pl.when(i == 0)
    def _():
        out_ref[...] = jnp.zeros((1, 1), jnp.float32)
        for b in range(_NBUF):
            _copy(b, b).start()

    cols = lax.broadcasted_iota(jnp.int32, (_ROWS, _V), 1)
    for b in range(_NBUF):
        blk = i * _NBUF + b
        _copy(blk, b).wait()
        xs = buf[b]
        tcol = t_ref[pl.ds(blk * _ROWS, _ROWS), :]
        mask = (tcol != 0).astype(jnp.float32)
        total = jnp.sum(xs * mask)
        col0 = jnp.sum(xs[:, 0:1] * mask)
        cnt = jnp.sum(mask)
        gsum = jnp.sum(jnp.where(cols == tcol, xs, 0.0) * mask)
        part = jnp.reshape(
            _K * cnt + _EPS * col0 - _EPS * total - _COEF * gsum, (1, 1)
        )
        out_ref[...] = out_ref[...] + part
        nxt = blk + _NBUF

        @pl.when(nxt < _NBLK)
        def _():
            _copy(nxt, b).start()


def _tc_sweep(x2, t2):
    return pl.pallas_call(
        _tc_body,
        grid=(_NSTEP,),
        in_specs=[
            pl.BlockSpec((_N, 1), lambda i: (0, 0)),
            pl.BlockSpec(memory_space=pltpu.MemorySpace.HBM),
        ],
        out_specs=pl.BlockSpec((1, 1), lambda i: (0, 0)),
        out_shape=jax.ShapeDtypeStruct((1, 1), jnp.float32),
        scratch_shapes=[
            pltpu.VMEM((_NBUF, _ROWS, _V), jnp.float32),
            pltpu.SemaphoreType.DMA((_NBUF,)),
        ],
    )(t2, x2)


def _lane_gather(v, idx):
    return v.at[idx].get(mode="promise_in_bounds")


def _all_lane_sum(v, lanes):
    for sh in (8, 4, 2, 1):
        v = v + _lane_gather(v, (lanes + sh) & 15)
    return v


def _row_sum(buf):
    def body(k, acc):
        s = None
        kbase = k * _CHUNK
        for m in range(_CHUNK // _LANES):
            v = buf[pl.ds(kbase + m * _LANES, _LANES)]
            s = v if s is None else s + v
        return acc + s

    return lax.fori_loop(0, _V // _CHUNK, body, jnp.zeros((_LANES,), jnp.float32))


def _sc_body(t_hbm, x_hbm, out_hbm, tgtd_v, buf0, buf1, res_v, sem0, sem1):
    wid = lax.axis_index("s") * 2 + lax.axis_index("c")
    lanes = lax.iota(jnp.int32, _LANES)

    dbase = _NTC + wid * _RPS
    ofs = dbase & 7
    pltpu.sync_copy(
        t_hbm.at[pl.ds(pl.multiple_of(dbase - ofs, 8), _NGRP * _LANES)],
        tgtd_v.at[pl.ds(0, _NGRP * _LANES)],
    )

    bufs = (buf0, buf1)
    sems = (sem0, sem1)

    def _rcopy(r, p):
        return pltpu.make_async_copy(x_hbm.at[dbase + r], bufs[p], sems[p])

    _rcopy(0, 0).start()
    _rcopy(1, 1).start()

    dense = jnp.zeros((_LANES,), jnp.float32)
    x0s = jnp.zeros((_LANES,), jnp.float32)
    cnts = jnp.zeros((_LANES,), jnp.float32)
    gath = jnp.zeros((_LANES,), jnp.float32)
    for g in range(_NGRP):
        jmax = min(_LANES, _RPS - g * _LANES)
        t16 = tgtd_v[pl.ds(ofs + g * _LANES, _LANES)]
        svec = jnp.zeros((_LANES,), jnp.float32)
        x0vec = jnp.zeros((_LANES,), jnp.float32)
        gvec = jnp.zeros((_LANES,), jnp.float32)
        for j in range(jmax):
            r = g * _LANES + j
            p = j & 1
            tr = t16[j]
            _rcopy(r, p).wait()
            head = bufs[p][pl.ds(0, _LANES)]
            win = bufs[p][pl.ds((tr >> 4) << 4, _LANES)]
            acc = _row_sum(bufs[p])
            if r + 2 < _RPS:
                _rcopy(r + 2, p).start()
            s = _all_lane_sum(acc, lanes)
            x0 = _lane_gather(head, jnp.zeros((_LANES,), jnp.int32))
            gval = _lane_gather(win, jnp.zeros((_LANES,), jnp.int32) + (tr & 15))
            svec = jnp.where(lanes == j, s, svec)
            x0vec = jnp.where(lanes == j, x0, x0vec)
            gvec = jnp.where(lanes == j, gval, gvec)
        m = (t16 != 0) & (lanes < jmax)
        dense = dense + jnp.where(m, svec, 0.0)
        x0s = x0s + jnp.where(m, x0vec, 0.0)
        cnts = cnts + jnp.where(m, 1.0, 0.0)
        gath = gath + jnp.where(m, gvec, 0.0)

    res_v[...] = _K * cnts + _EPS * x0s - _EPS * dense - _COEF * gath
    pltpu.sync_copy(res_v, out_hbm.at[wid])


@functools.lru_cache(maxsize=None)
def _sc_kernel():
    return pl.kernel(
        _sc_body,
        out_type=jax.ShapeDtypeStruct((_NW, _LANES), jnp.float32),
        mesh=plsc.VectorSubcoreMesh(core_axis_name="c", subcore_axis_name="s"),
        scratch_types=[
            pltpu.VMEM((_NGRP * _LANES + 8,), jnp.int32),
            pltpu.VMEM((_V,), jnp.float32),
            pltpu.VMEM((_V,), jnp.float32),
            pltpu.VMEM((_LANES,), jnp.float32),
            pltpu.SemaphoreType.DMA,
            pltpu.SemaphoreType.DMA,
        ],
    )


@jax.jit
def kernel(x, target):
    t = target.reshape(-1).astype(jnp.int32)
    x2 = x.reshape(_N, _V)
    sc_part = _sc_kernel()(t, x2)
    tc_part = _tc_sweep(x2, t.reshape(_N, 1))
    return tc_part[0, 0] + jnp.sum(sc_part)
